# baseline (speedup 1.0000x reference)
import functools

import jax
import jax.numpy as jnp
from jax import lax
from jax.experimental import pallas as pl
from jax.experimental.pallas import tpu as pltpu
from jax.experimental.pallas import tpu_sc as plsc

N = 10000
E = 320000
F = 128
NPAD = 10240
EPAD = 327680
NTILES = 16
RPT = NPAD // NTILES
EPT = EPAD // NTILES
CH = 128
NCHUNK = EPT // CH
SLABS = 4
CPS = NCHUNK // SLABS
LANES = 16


def _sc_degrees_body(idx2, wflat, zflat, deg2,
                     iall, wall, acc1d, stage_buf, res, shared, sem):
    c = lax.axis_index("c")
    s = lax.axis_index("s")
    pltpu.sync_copy(zflat, acc1d)
    pltpu.sync_copy(idx2.at[c, pl.ds(s * EPT, EPT)], iall)
    pltpu.sync_copy(wflat.at[pl.ds(s * EPT, EPT)], wall)

    UNROLL = 8

    def body(k, carry):
        base = k * (LANES * UNROLL)
        for j in range(UNROLL):
            plsc.addupdate_scatter(
                acc1d, [iall[pl.ds(base + j * LANES, LANES)]],
                wall[pl.ds(base + j * LANES, LANES)])
        return carry

    lax.fori_loop(0, EPT // (LANES * UNROLL), body, 0)

    pltpu.sync_copy(acc1d, shared.at[s, :])
    plsc.subcore_barrier()
    for r in range(NTILES):
        pltpu.sync_copy(shared.at[r, pl.ds(s * RPT, RPT)],
                        stage_buf.at[pl.ds(r * RPT, RPT)])

    def red_body(j, carry):
        tot = stage_buf[pl.ds(j * LANES, LANES)]
        for r in range(1, NTILES):
            tot = tot + stage_buf[pl.ds(r * RPT + j * LANES, LANES)]
        res[pl.ds(j * LANES, LANES)] = tot
        return carry

    lax.fori_loop(0, RPT // LANES, red_body, 0)
    pltpu.sync_copy(res, deg2.at[c, pl.ds(s * RPT, RPT)])


def _make_spmm_body(stacked_src):
    def body(src_tab, deg2, gidx3, sidx4, zeros128, btab, t12,
             gall, sall, rows0, rows1, dbuf, acc, sem0, sem1):
        c = lax.axis_index("c")
        s = lax.axis_index("s")
        pltpu.sync_copy(zeros128, rows0)
        for r in range(RPT // CH):
            pltpu.sync_copy(rows0, acc.at[pl.ds(s * RPT + r * CH, CH), :])

        pltpu.sync_copy(deg2.at[c, pl.ds(s * RPT, RPT)], dbuf)

        def rcp_body(j, carry):
            d = dbuf[pl.ds(j * LANES, LANES)]
            dbuf[pl.ds(j * LANES, LANES)] = 1.0 / jnp.maximum(d, 1e-12)
            return carry

        lax.fori_loop(0, RPT // LANES, rcp_body, 0)

        row0 = s * RPT

        def scale_body(ch, carry):
            r0 = row0 + ch * CH
            if stacked_src:
                pltpu.sync_copy(src_tab.at[c, pl.ds(r0, CH), :], rows0)
            else:
                pltpu.sync_copy(src_tab.at[pl.ds(r0, CH), :], rows0)
            for g in range(CH // LANES):
                grp = dbuf[pl.ds(ch * CH + g * LANES, LANES)]
                for l in range(LANES):
                    b = jnp.take_along_axis(
                        grp, jnp.full((LANES,), l, jnp.int32), axis=0)
                    row = g * LANES + l
                    for cb in range(F // LANES):
                        v = rows0[row, pl.ds(cb * LANES, LANES)]
                        rows0[row, pl.ds(cb * LANES, LANES)] = v * b
            pltpu.sync_copy(rows0, btab.at[c, pl.ds(r0, CH), :])
            return carry

        lax.fori_loop(0, RPT // CH, scale_body, 0)
        plsc.subcore_barrier()

        def run(table):
            HF = CH // 2

            def fire(i, buf, sem):
                pltpu.async_copy(table.at[gall.at[pl.ds(i * CH, HF)]],
                                 buf.at[pl.ds(0, HF), :], sem)
                pltpu.async_copy(table.at[gall.at[pl.ds(i * CH + HF, HF)]],
                                 buf.at[pl.ds(HF, HF), :], sem)

            def slab_body(t, carry0):
                pltpu.sync_copy(
                    gidx3.at[c, s, pl.ds(t * CPS * CH, CPS * CH)], gall)
                pltpu.sync_copy(sidx4.at[c, s, pl.ds(t * CPS, CPS)], sall)
                fire(0, rows0, sem0)
                fire(1, rows1, sem1)

                def body2(k, carry):
                    i0 = 2 * k
                    pltpu.make_async_copy(table.at[gall.at[pl.ds(0, CH)]],
                                          rows0, sem0).wait()
                    pltpu.sync_copy(rows0, acc.at[sall.at[i0]], add=True)

                    @pl.when(i0 + 2 < CPS)
                    def _():
                        fire(i0 + 2, rows0, sem0)

                    pltpu.make_async_copy(table.at[gall.at[pl.ds(0, CH)]],
                                          rows1, sem1).wait()
                    pltpu.sync_copy(rows1, acc.at[sall.at[i0 + 1]], add=True)

                    @pl.when(i0 + 3 < CPS)
                    def _():
                        fire(i0 + 3, rows1, sem1)

                    return carry

                lax.fori_loop(0, CPS // 2, body2, 0)
                return carry0

            lax.fori_loop(0, SLABS, slab_body, 0)

        @pl.when(c == 0)
        def _():
            run(btab.at[0])

        @pl.when(c == 1)
        def _():
            run(btab.at[1])

        plsc.subcore_barrier()
        pltpu.sync_copy(acc.at[pl.ds(s * RPT, RPT), :],
                        t12.at[c, pl.ds(s * RPT, RPT), :])

    return body


@functools.lru_cache(maxsize=None)
def _sc_kernels():
    mesh = plsc.VectorSubcoreMesh(core_axis_name="c", subcore_axis_name="s")
    deg = pl.kernel(
        _sc_degrees_body,
        out_type=jax.ShapeDtypeStruct((2, NPAD), jnp.float32),
        mesh=mesh,
        scratch_types=[
            pltpu.VMEM((EPT,), jnp.int32),
            pltpu.VMEM((EPT,), jnp.float32),
            pltpu.VMEM((NPAD,), jnp.float32),
            pltpu.VMEM((NTILES * RPT,), jnp.float32),
            pltpu.VMEM((RPT,), jnp.float32),
            pltpu.VMEM_SHARED((NTILES, NPAD), jnp.float32),
            pltpu.SemaphoreType.DMA,
        ],
        compiler_params=pltpu.CompilerParams(needs_layout_passes=False),
    )

    def make_spmm(stacked_src):
        return pl.kernel(
            _make_spmm_body(stacked_src),
            out_type=[
                jax.ShapeDtypeStruct((2, NPAD, F), jnp.float32),
                jax.ShapeDtypeStruct((2, NPAD, F), jnp.float32),
            ],
            mesh=mesh,
            scratch_types=[
                pltpu.VMEM((CPS * CH,), jnp.int32),
                pltpu.VMEM((CPS, CH), jnp.int32),
                pltpu.VMEM((CH, F), jnp.float32),
                pltpu.VMEM((CH, F), jnp.float32),
                pltpu.VMEM((RPT,), jnp.float32),
                pltpu.VMEM_SHARED((NPAD, F), jnp.float32),
                pltpu.SemaphoreType.DMA,
                pltpu.SemaphoreType.DMA,
            ],
        )

    return deg, make_spmm(False), make_spmm(True)


_ROWS = 1024


def _final_body(x_ref, b1o_ref, b1i_ref, p2o_ref, p2i_ref, degt_ref,
                wz_ref, wh_ref, bz_ref, bh_ref, wcls_ref, bcls_ref,
                out_ref):
    xb = x_ref[...]
    b1o = b1o_ref[...]
    b1i = b1i_ref[...]
    p2o = p2o_ref[...]
    p2i = p2i_ref[...]
    do = jnp.maximum(degt_ref[...][:, 0:1], 1e-12)
    di = jnp.maximum(degt_ref[...][:, 1:2], 1e-12)

    def conv(W, b):
        wx = W[0, 0, :F] + W[1, 0, :F] - W[0, 2, :F] - W[1, 2, :F]
        h = jnp.dot(xb, wx, preferred_element_type=jnp.float32)
        h += jnp.dot(b1o, W[0, 1, :F], preferred_element_type=jnp.float32) * do
        h += jnp.dot(b1i, W[1, 1, :F], preferred_element_type=jnp.float32) * di
        h += 2.0 * jnp.dot(p2o, W[0, 2, :F], preferred_element_type=jnp.float32)
        h += 2.0 * jnp.dot(p2i, W[1, 2, :F], preferred_element_type=jnp.float32)
        return h + b

    z = jax.nn.sigmoid(conv(wz_ref[...], bz_ref[...]))
    ht = jnp.tanh(conv(wh_ref[...], bh_ref[...]))
    act = jax.nn.relu((1.0 - z) * ht)
    out_ref[...] = (jnp.dot(act, wcls_ref[...], preferred_element_type=jnp.float32)
                    + bcls_ref[...])


def _final(x_pad, b1o, b1i, p2o, p2i, degt, W_z, W_h, b_z, b_h, W_cls, b_cls):
    grid = NPAD // _ROWS
    row_spec = pl.BlockSpec((_ROWS, F), lambda i: (i, 0))
    return pl.pallas_call(
        _final_body,
        grid=(grid,),
        in_specs=[
            row_spec, row_spec, row_spec, row_spec, row_spec,
            pl.BlockSpec((_ROWS, 2), lambda i: (i, 0)),
            pl.BlockSpec((2, 3, 2 * F, F), lambda i: (0, 0, 0, 0)),
            pl.BlockSpec((2, 3, 2 * F, F), lambda i: (0, 0, 0, 0)),
            pl.BlockSpec((1, F), lambda i: (0, 0)),
            pl.BlockSpec((1, F), lambda i: (0, 0)),
            pl.BlockSpec((F, 1), lambda i: (0, 0)),
            pl.BlockSpec((1, 1), lambda i: (0, 0)),
        ],
        out_specs=pl.BlockSpec((_ROWS, 1), lambda i: (i, 0)),
        out_shape=jax.ShapeDtypeStruct((NPAD, 1), jnp.float32),
    )(x_pad, b1o, b1i, p2o, p2i, degt, W_z, W_h, b_z, b_h, W_cls, b_cls)


def kernel(x, edge_index, edge_weight, W_z, b_z, W_r, b_r, W_h, b_h,
           W_cls, b_cls):
    del W_r, b_r
    x_pad = jnp.pad(x, ((0, NPAD - N), (0, 0)))
    pad_idx = jnp.full((EPAD - E,), NPAD - 1, jnp.int32)
    srcp = jnp.concatenate([edge_index[0], pad_idx])
    dstp = jnp.concatenate([edge_index[1], pad_idx])
    wflat = jnp.pad(edge_weight, (0, EPAD - E))
    zflat = jnp.zeros((NPAD,), jnp.float32)
    zeros128 = jnp.zeros((CH, F), jnp.float32)
    idx2 = jnp.stack([srcp, dstp])
    gidx3 = idx2.reshape(2, NTILES, EPT)
    sidx4 = jnp.stack([dstp, srcp]).reshape(2, NTILES, NCHUNK, CH)

    sc_degrees, sc_spmm1, sc_spmm2 = _sc_kernels()
    deg2 = sc_degrees(idx2, wflat, zflat)
    _, t12 = sc_spmm1(x_pad, deg2, gidx3, sidx4, zeros128)
    b12, p12 = sc_spmm2(t12, deg2, gidx3, sidx4, zeros128)

    out = _final(x_pad, b12[0], b12[1], p12[0], p12[1], deg2.T,
                 W_z, W_h, b_z.reshape(1, F), b_h.reshape(1, F),
                 W_cls, b_cls.reshape(1, 1))
    return out[:N]


# speedup vs baseline: 1.0970x; 1.0970x over previous
import functools

import jax
import jax.numpy as jnp
from jax import lax
from jax.experimental import pallas as pl
from jax.experimental.pallas import tpu as pltpu
from jax.experimental.pallas import tpu_sc as plsc

N = 10000
E = 320000
F = 128
NPAD = 10240
EPAD = 327680
NTILES = 16
RPT = NPAD // NTILES
EPT = EPAD // NTILES
CH = 128
NCHUNK = EPT // CH
SLABS = 4
CPS = NCHUNK // SLABS
LANES = 16


def _sc_degrees_body(idx2, wflat, zflat, deg2,
                     iall, wall, acc1d, stage_buf, res, shared, sem):
    c = lax.axis_index("c")
    s = lax.axis_index("s")
    pltpu.sync_copy(zflat, acc1d)
    pltpu.sync_copy(idx2.at[c, pl.ds(s * EPT, EPT)], iall)
    pltpu.sync_copy(wflat.at[pl.ds(s * EPT, EPT)], wall)

    UNROLL = 8

    def body(k, carry):
        base = k * (LANES * UNROLL)
        for j in range(UNROLL):
            plsc.addupdate_scatter(
                acc1d, [iall[pl.ds(base + j * LANES, LANES)]],
                wall[pl.ds(base + j * LANES, LANES)])
        return carry

    lax.fori_loop(0, EPT // (LANES * UNROLL), body, 0)

    pltpu.sync_copy(acc1d, shared.at[s, :])
    plsc.subcore_barrier()
    for r in range(NTILES):
        pltpu.sync_copy(shared.at[r, pl.ds(s * RPT, RPT)],
                        stage_buf.at[pl.ds(r * RPT, RPT)])

    def red_body(j, carry):
        tot = stage_buf[pl.ds(j * LANES, LANES)]
        for r in range(1, NTILES):
            tot = tot + stage_buf[pl.ds(r * RPT + j * LANES, LANES)]
        res[pl.ds(j * LANES, LANES)] = tot
        return carry

    lax.fori_loop(0, RPT // LANES, red_body, 0)
    pltpu.sync_copy(res, deg2.at[c, pl.ds(s * RPT, RPT)])


def _sc_spmm_body(a_o, a_i, gidx3, sidx4, zeros128, t12,
                  gall, sall, rows0, rows1, acc, sem0, sem1):
        c = lax.axis_index("c")
        s = lax.axis_index("s")
        pltpu.sync_copy(zeros128, rows0)
        for r in range(RPT // CH):
            pltpu.sync_copy(rows0, acc.at[pl.ds(s * RPT + r * CH, CH), :])
        plsc.subcore_barrier()

        def run(table):
            HF = CH // 2

            def fire(i, buf, sem):
                pltpu.async_copy(table.at[gall.at[pl.ds(i * CH, HF)]],
                                 buf.at[pl.ds(0, HF), :], sem)
                pltpu.async_copy(table.at[gall.at[pl.ds(i * CH + HF, HF)]],
                                 buf.at[pl.ds(HF, HF), :], sem)

            def slab_body(t, carry0):
                pltpu.sync_copy(
                    gidx3.at[c, s, pl.ds(t * CPS * CH, CPS * CH)], gall)
                pltpu.sync_copy(sidx4.at[c, s, pl.ds(t * CPS, CPS)], sall)
                fire(0, rows0, sem0)
                fire(1, rows1, sem1)

                def body2(k, carry):
                    i0 = 2 * k
                    pltpu.make_async_copy(table.at[gall.at[pl.ds(0, CH)]],
                                          rows0, sem0).wait()
                    pltpu.sync_copy(rows0, acc.at[sall.at[i0]], add=True)

                    @pl.when(i0 + 2 < CPS)
                    def _():
                        fire(i0 + 2, rows0, sem0)

                    pltpu.make_async_copy(table.at[gall.at[pl.ds(0, CH)]],
                                          rows1, sem1).wait()
                    pltpu.sync_copy(rows1, acc.at[sall.at[i0 + 1]], add=True)

                    @pl.when(i0 + 3 < CPS)
                    def _():
                        fire(i0 + 3, rows1, sem1)

                    return carry

                lax.fori_loop(0, CPS // 2, body2, 0)
                return carry0

            lax.fori_loop(0, SLABS, slab_body, 0)

        @pl.when(c == 0)
        def _():
            run(a_o)

        @pl.when(c == 1)
        def _():
            run(a_i)

        plsc.subcore_barrier()
        pltpu.sync_copy(acc.at[pl.ds(s * RPT, RPT), :],
                        t12.at[c, pl.ds(s * RPT, RPT), :])


@functools.lru_cache(maxsize=None)
def _sc_kernels():
    mesh = plsc.VectorSubcoreMesh(core_axis_name="c", subcore_axis_name="s")
    deg = pl.kernel(
        _sc_degrees_body,
        out_type=jax.ShapeDtypeStruct((2, NPAD), jnp.float32),
        mesh=mesh,
        scratch_types=[
            pltpu.VMEM((EPT,), jnp.int32),
            pltpu.VMEM((EPT,), jnp.float32),
            pltpu.VMEM((NPAD,), jnp.float32),
            pltpu.VMEM((NTILES * RPT,), jnp.float32),
            pltpu.VMEM((RPT,), jnp.float32),
            pltpu.VMEM_SHARED((NTILES, NPAD), jnp.float32),
            pltpu.SemaphoreType.DMA,
        ],
        compiler_params=pltpu.CompilerParams(needs_layout_passes=False),
    )

    spmm = pl.kernel(
        _sc_spmm_body,
        out_type=jax.ShapeDtypeStruct((2, NPAD, F), jnp.float32),
        mesh=mesh,
        scratch_types=[
            pltpu.VMEM((CPS * CH,), jnp.int32),
            pltpu.VMEM((CPS, CH), jnp.int32),
            pltpu.VMEM((CH, F), jnp.float32),
            pltpu.VMEM((CH, F), jnp.float32),
            pltpu.VMEM_SHARED((NPAD, F), jnp.float32),
            pltpu.SemaphoreType.DMA,
            pltpu.SemaphoreType.DMA,
        ],
    )

    return deg, spmm


_ROWS = 1024


def _prescale_body(vo_ref, vi_ref, degt_ref, ao_ref, ai_ref):
    ro = 1.0 / jnp.maximum(degt_ref[...][:, 0:1], 1e-12)
    ri = 1.0 / jnp.maximum(degt_ref[...][:, 1:2], 1e-12)
    ao_ref[...] = vo_ref[...] * ro
    ai_ref[...] = vi_ref[...] * ri


def _prescale(v_o, v_i, degt):
    grid = NPAD // _ROWS
    row_spec = pl.BlockSpec((_ROWS, F), lambda i: (i, 0))
    return pl.pallas_call(
        _prescale_body,
        grid=(grid,),
        in_specs=[
            row_spec, row_spec,
            pl.BlockSpec((_ROWS, 2), lambda i: (i, 0)),
        ],
        out_specs=[row_spec, row_spec],
        out_shape=[
            jax.ShapeDtypeStruct((NPAD, F), jnp.float32),
            jax.ShapeDtypeStruct((NPAD, F), jnp.float32),
        ],
    )(v_o, v_i, degt)


def _final_body(x_ref, t1o_ref, t1i_ref, p2o_ref, p2i_ref,
                wz_ref, wh_ref, bz_ref, bh_ref, wcls_ref, bcls_ref,
                out_ref):
    xb = x_ref[...]
    t1o = t1o_ref[...]
    t1i = t1i_ref[...]
    p2o = p2o_ref[...]
    p2i = p2i_ref[...]

    def conv(W, b):
        wx = W[0, 0, :F] + W[1, 0, :F] - W[0, 2, :F] - W[1, 2, :F]
        h = jnp.dot(xb, wx, preferred_element_type=jnp.float32)
        h += jnp.dot(t1o, W[0, 1, :F], preferred_element_type=jnp.float32)
        h += jnp.dot(t1i, W[1, 1, :F], preferred_element_type=jnp.float32)
        h += 2.0 * jnp.dot(p2o, W[0, 2, :F], preferred_element_type=jnp.float32)
        h += 2.0 * jnp.dot(p2i, W[1, 2, :F], preferred_element_type=jnp.float32)
        return h + b

    z = jax.nn.sigmoid(conv(wz_ref[...], bz_ref[...]))
    ht = jnp.tanh(conv(wh_ref[...], bh_ref[...]))
    act = jax.nn.relu((1.0 - z) * ht)
    out_ref[...] = (jnp.dot(act, wcls_ref[...], preferred_element_type=jnp.float32)
                    + bcls_ref[...])


def _final(x_pad, t1o, t1i, p2o, p2i, W_z, W_h, b_z, b_h, W_cls, b_cls):
    grid = NPAD // _ROWS
    row_spec = pl.BlockSpec((_ROWS, F), lambda i: (i, 0))
    return pl.pallas_call(
        _final_body,
        grid=(grid,),
        in_specs=[
            row_spec, row_spec, row_spec, row_spec, row_spec,
            pl.BlockSpec((2, 3, 2 * F, F), lambda i: (0, 0, 0, 0)),
            pl.BlockSpec((2, 3, 2 * F, F), lambda i: (0, 0, 0, 0)),
            pl.BlockSpec((1, F), lambda i: (0, 0)),
            pl.BlockSpec((1, F), lambda i: (0, 0)),
            pl.BlockSpec((F, 1), lambda i: (0, 0)),
            pl.BlockSpec((1, 1), lambda i: (0, 0)),
        ],
        out_specs=pl.BlockSpec((_ROWS, 1), lambda i: (i, 0)),
        out_shape=jax.ShapeDtypeStruct((NPAD, 1), jnp.float32),
    )(x_pad, t1o, t1i, p2o, p2i, W_z, W_h, b_z, b_h, W_cls, b_cls)


def kernel(x, edge_index, edge_weight, W_z, b_z, W_r, b_r, W_h, b_h,
           W_cls, b_cls):
    del W_r, b_r
    x_pad = jnp.pad(x, ((0, NPAD - N), (0, 0)))
    pad_idx = jnp.full((EPAD - E,), NPAD - 1, jnp.int32)
    srcp = jnp.concatenate([edge_index[0], pad_idx])
    dstp = jnp.concatenate([edge_index[1], pad_idx])
    wflat = jnp.pad(edge_weight, (0, EPAD - E))
    zflat = jnp.zeros((NPAD,), jnp.float32)
    zeros128 = jnp.zeros((CH, F), jnp.float32)
    idx2 = jnp.stack([srcp, dstp])
    gidx3 = idx2.reshape(2, NTILES, EPT)
    sidx4 = jnp.stack([dstp, srcp]).reshape(2, NTILES, NCHUNK, CH)

    sc_degrees, sc_spmm = _sc_kernels()
    deg2 = sc_degrees(idx2, wflat, zflat)
    degt = deg2.T
    a_o, a_i = _prescale(x_pad, x_pad, degt)
    t12 = sc_spmm(a_o, a_i, gidx3, sidx4, zeros128)
    b_o, b_i = _prescale(t12[0], t12[1], degt)
    p12 = sc_spmm(b_o, b_i, gidx3, sidx4, zeros128)

    out = _final(x_pad, t12[0], t12[1], p12[0], p12[1],
                 W_z, W_h, b_z.reshape(1, F), b_h.reshape(1, F),
                 W_cls, b_cls.reshape(1, 1))
    return out[:N]
